# baseline (speedup 1.0000x reference)
import functools

import jax
import jax.numpy as jnp
from jax import lax
from jax.experimental import pallas as pl
from jax.experimental.pallas import tpu as pltpu
from jax.experimental.pallas import tpu_sc as plsc

NC = 2
NS = 16
K = 256
NBUF = 2
D = 4


def _ceil_to(x, m):
  return ((x + m - 1) // m) * m


def _embed_body(f_ref, w_ref, b_ref, o_ref):
  h = jnp.dot(f_ref[...], w_ref[...], preferred_element_type=jnp.float32)
  o_ref[...] = jnp.maximum(h + b_ref[...], 0.0)


def _embed(feat2, w2, b2, bnp):
  n2, fi2 = feat2.shape
  return pl.pallas_call(
      _embed_body,
      grid=(n2 // bnp,),
      in_specs=[
          pl.BlockSpec((bnp, fi2), lambda i: (i, 0)),
          pl.BlockSpec((fi2, 128), lambda i: (0, 0)),
          pl.BlockSpec((1, 128), lambda i: (0, 0)),
      ],
      out_specs=pl.BlockSpec((bnp, 128), lambda i: (i, 0)),
      out_shape=jax.ShapeDtypeStruct((n2, 128), jnp.float32),
  )(feat2, w2, b2.reshape(1, 128))


def _update_body(h_ref, m_ref, w_ref, b_ref, o_ref):
  o = jnp.dot(m_ref[...], w_ref[...], preferred_element_type=jnp.float32)
  o_ref[...] = jnp.maximum(h_ref[...] + o + b_ref[...], 0.0)


def _update(h2, msgs2, w2, b2, bnp):
  n2 = h2.shape[0]
  return pl.pallas_call(
      _update_body,
      grid=(n2 // bnp,),
      in_specs=[
          pl.BlockSpec((bnp, 128), lambda i: (i, 0)),
          pl.BlockSpec((bnp, 128), lambda i: (i, 0)),
          pl.BlockSpec((128, 128), lambda i: (0, 0)),
          pl.BlockSpec((1, 128), lambda i: (0, 0)),
      ],
      out_specs=pl.BlockSpec((bnp, 128), lambda i: (i, 0)),
      out_shape=jax.ShapeDtypeStruct((n2, 128), jnp.float32),
  )(h2, msgs2, w2, b2.reshape(1, 128))


def _score_body(h_ref, m_ref, w_ref, b_ref, ws_ref, bs_ref, o_ref):
  o = jnp.dot(m_ref[...], w_ref[...], preferred_element_type=jnp.float32)
  o = jnp.maximum(h_ref[...] + o + b_ref[...], 0.0)
  o_ref[...] = jnp.dot(o, ws_ref[...], preferred_element_type=jnp.float32) + bs_ref[...]


def _score(h2, msgs2, w2, b2, ws2, bs2, bnp):
  n2 = h2.shape[0]
  return pl.pallas_call(
      _score_body,
      grid=(n2 // bnp,),
      in_specs=[
          pl.BlockSpec((bnp, 128), lambda i: (i, 0)),
          pl.BlockSpec((bnp, 128), lambda i: (i, 0)),
          pl.BlockSpec((128, 128), lambda i: (0, 0)),
          pl.BlockSpec((1, 128), lambda i: (0, 0)),
          pl.BlockSpec((128, 2), lambda i: (0, 0)),
          pl.BlockSpec((1, 2), lambda i: (0, 0)),
      ],
      out_specs=pl.BlockSpec((bnp, 2), lambda i: (i, 0)),
      out_shape=jax.ShapeDtypeStruct((n2, 2), jnp.float32),
  )(h2, msgs2, w2, b2.reshape(1, 128), ws2, bs2.reshape(1, 2))


@functools.cache
def _make_sc_pass(n_src2, n_dst, n_dst_pad, nchunk_tot):
  nchunk_t = nchunk_tot // NS
  rows_per_tile = n_dst_pad // NS
  dr0 = n_dst // NS
  dr_last = n_dst - dr0 * (NS - 1)
  assert nchunk_t % D == 0
  mesh = plsc.VectorSubcoreMesh(core_axis_name="c", subcore_axis_name="s")

  @functools.partial(
      pl.kernel,
      out_type=jax.ShapeDtypeStruct((n_dst, 64), jnp.float32),
      mesh=mesh,
      scratch_types=[
          pltpu.VMEM_SHARED((n_dst_pad, 32), jnp.float32),
          pltpu.VMEM((D, 2, K), jnp.int32),
          pltpu.VMEM((NBUF, K, 32), jnp.float32),
          [pltpu.SemaphoreType.DMA] * D,
          [pltpu.SemaphoreType.DMA] * NBUF,
      ],
      compiler_params=pltpu.CompilerParams(use_tc_tiling_on_sc=False),
  )
  def sc_pass(t_hbm, sidx_hbm, didx_hbm, zeros_hbm, out_hbm, accum, idx_v,
              rows_v, isem, gsem):
    c = lax.axis_index("c")
    s = lax.axis_index("s")
    row0 = s * nchunk_t

    def _ifetch(row, u):
      pltpu.async_copy(sidx_hbm.at[row], idx_v.at[u].at[0], isem[u])
      pltpu.async_copy(didx_hbm.at[row], idx_v.at[u].at[1], isem[u])

    def _iwait(row, u):
      pltpu.make_async_copy(sidx_hbm.at[row], idx_v.at[u].at[0], isem[u]).wait()
      pltpu.make_async_copy(didx_hbm.at[row], idx_v.at[u].at[1], isem[u]).wait()
    t_half = t_hbm.at[pl.ds(c, n_src2 - 1)]

    pltpu.sync_copy(zeros_hbm.at[pl.ds(s * rows_per_tile, rows_per_tile)],
                    accum.at[pl.ds(s * rows_per_tile, rows_per_tile)])

    for u in range(D):
      _ifetch(row0 + u, u)
    for u in range(NBUF):
      _iwait(row0 + u, u)
      pltpu.async_copy(t_half.at[idx_v.at[u].at[0]], rows_v.at[u], gsem[u])

    plsc.subcore_barrier()

    def inner(jj, carry):
      base = jj * D
      for u in range(D):
        j = base + u
        b = u % NBUF
        un = (u + NBUF) % D
        pltpu.make_async_copy(
            t_half.at[idx_v.at[u].at[0]], rows_v.at[b], gsem[b]).wait()
        pltpu.sync_copy(rows_v.at[b], accum.at[idx_v.at[u].at[1]], add=True)
        @pl.when(j + D < nchunk_t)
        def _refill():
          _ifetch(row0 + j + D, u)
        @pl.when(j + NBUF < nchunk_t)
        def _issue():
          _iwait(row0 + j + NBUF, un)
          pltpu.async_copy(
              t_half.at[idx_v.at[un].at[0]], rows_v.at[b], gsem[b])
      return carry
    lax.fori_loop(0, nchunk_t // D, inner, 0)

    plsc.subcore_barrier()

    def _drain(sl):
      @pl.when(c == 0)
      def _d0():
        pltpu.sync_copy(accum.at[sl], out_hbm.at[sl, pl.ds(0, 32)])

      @pl.when(c == 1)
      def _d1():
        pltpu.sync_copy(accum.at[sl], out_hbm.at[sl, pl.ds(32, 32)])

    if dr0 * NS == n_dst:
      _drain(pl.ds(s * dr0, dr0))
    else:
      @pl.when(s < NS - 1)
      def _not_last():
        _drain(pl.ds(s * dr0, dr0))

      @pl.when(s == NS - 1)
      def _last():
        _drain(pl.ds((NS - 1) * dr0, dr_last))

  return sc_pass


def kernel(var_feat, constr_feat, edge_index_var_to_constr,
           W_var, b_var, W_constr, b_constr,
           W_v2c, b_v2c, W_c2v, b_c2v, W_score, b_score):
  v = var_feat.shape[0]
  cn = constr_feat.shape[0]
  e = edge_index_var_to_constr.shape[1]

  v_pad = _ceil_to(v + 1, NS * K)
  c_pad = _ceil_to(cn + 1, NS * K)
  e_pad = _ceil_to(e + 1, NS * K * D)
  nchunk_tot = e_pad // K

  eidx = edge_index_var_to_constr.astype(jnp.int32)
  vidx, cidx = eidx[0], eidx[1]
  npad = e_pad - e
  ar = jnp.arange(npad, dtype=jnp.int32)
  sidx_v2c = (2 * jnp.concatenate([vidx, ar % v])).reshape(nchunk_tot, K)
  didx_v2c = jnp.concatenate([cidx, cn + ar % (c_pad - cn)]).reshape(nchunk_tot, K)
  sidx_c2v = (2 * jnp.concatenate([cidx, ar % cn])).reshape(nchunk_tot, K)
  didx_c2v = jnp.concatenate([vidx, v + ar % (v_pad - v)]).reshape(nchunk_tot, K)

  v2c = _make_sc_pass(2 * v, cn, c_pad, nchunk_tot)
  c2v = _make_sc_pass(2 * cn, v, v_pad, nchunk_tot)

  eye2 = jnp.eye(2, dtype=jnp.float32)
  w_var2 = jnp.kron(eye2, W_var)
  w_constr2 = jnp.kron(eye2, W_constr)
  w_v2c2 = jnp.kron(eye2, W_v2c)
  w_c2v2 = jnp.kron(eye2, W_c2v)
  ws2 = jnp.kron(eye2, W_score)
  b_var2 = jnp.tile(b_var, 2)
  b_constr2 = jnp.tile(b_constr, 2)
  b_v2c2 = jnp.tile(b_v2c, 2)
  b_c2v2 = jnp.tile(b_c2v, 2)
  bs2 = jnp.tile(b_score, 2)

  zeros = jnp.zeros((v_pad, 32), jnp.float32)

  h_var = _embed(var_feat.reshape(v // 2, 256), w_var2, b_var2, 5000)
  h_constr = _embed(constr_feat.reshape(cn // 2, 256), w_constr2, b_constr2,
                    cn // 2)

  rounds = 3
  for r in range(rounds):
    msgs_c = v2c(h_var.reshape(2 * v, 32), sidx_v2c, didx_v2c, zeros)
    h_constr = _update(h_constr, msgs_c.reshape(cn // 2, 128),
                       w_v2c2, b_v2c2, cn // 2)
    msgs_v = c2v(h_constr.reshape(2 * cn, 32), sidx_c2v, didx_c2v, zeros)
    if r < rounds - 1:
      h_var = _update(h_var, msgs_v.reshape(v // 2, 128),
                      w_c2v2, b_c2v2, 5000)
    else:
      scores = _score(h_var, msgs_v.reshape(v // 2, 128),
                      w_c2v2, b_c2v2, ws2, bs2, 5000)

  return scores.reshape(-1)


# speedup vs baseline: 1.1545x; 1.1545x over previous
import functools

import jax
import jax.numpy as jnp
from jax import lax
from jax.experimental import pallas as pl
from jax.experimental.pallas import tpu as pltpu
from jax.experimental.pallas import tpu_sc as plsc

NC = 2
NS = 16
K = 128
NBUF = 4
D = 8


def _ceil_to(x, m):
  return ((x + m - 1) // m) * m


def _embed_body(f_ref, w_ref, b_ref, o_ref):
  h = jnp.dot(f_ref[...], w_ref[...], preferred_element_type=jnp.float32)
  o_ref[...] = jnp.maximum(h + b_ref[...], 0.0)


def _embed(feat2, w2, b2, bnp):
  n2, fi2 = feat2.shape
  return pl.pallas_call(
      _embed_body,
      grid=(n2 // bnp,),
      in_specs=[
          pl.BlockSpec((bnp, fi2), lambda i: (i, 0)),
          pl.BlockSpec((fi2, 128), lambda i: (0, 0)),
          pl.BlockSpec((1, 128), lambda i: (0, 0)),
      ],
      out_specs=pl.BlockSpec((bnp, 128), lambda i: (i, 0)),
      out_shape=jax.ShapeDtypeStruct((n2, 128), jnp.float32),
  )(feat2, w2, b2.reshape(1, 128))


def _update_body(h_ref, m_ref, w_ref, b_ref, o_ref):
  o = jnp.dot(m_ref[...], w_ref[...], preferred_element_type=jnp.float32)
  o_ref[...] = jnp.maximum(h_ref[...] + o + b_ref[...], 0.0)


def _update(h2, msgs2, w2, b2, bnp):
  n2 = h2.shape[0]
  return pl.pallas_call(
      _update_body,
      grid=(n2 // bnp,),
      in_specs=[
          pl.BlockSpec((bnp, 128), lambda i: (i, 0)),
          pl.BlockSpec((bnp, 128), lambda i: (i, 0)),
          pl.BlockSpec((128, 128), lambda i: (0, 0)),
          pl.BlockSpec((1, 128), lambda i: (0, 0)),
      ],
      out_specs=pl.BlockSpec((bnp, 128), lambda i: (i, 0)),
      out_shape=jax.ShapeDtypeStruct((n2, 128), jnp.float32),
  )(h2, msgs2, w2, b2.reshape(1, 128))


def _score_body(h_ref, m_ref, w_ref, b_ref, ws_ref, bs_ref, o_ref):
  o = jnp.dot(m_ref[...], w_ref[...], preferred_element_type=jnp.float32)
  o = jnp.maximum(h_ref[...] + o + b_ref[...], 0.0)
  o_ref[...] = jnp.dot(o, ws_ref[...], preferred_element_type=jnp.float32) + bs_ref[...]


def _score(h2, msgs2, w2, b2, ws2, bs2, bnp):
  n2 = h2.shape[0]
  return pl.pallas_call(
      _score_body,
      grid=(n2 // bnp,),
      in_specs=[
          pl.BlockSpec((bnp, 128), lambda i: (i, 0)),
          pl.BlockSpec((bnp, 128), lambda i: (i, 0)),
          pl.BlockSpec((128, 128), lambda i: (0, 0)),
          pl.BlockSpec((1, 128), lambda i: (0, 0)),
          pl.BlockSpec((128, 2), lambda i: (0, 0)),
          pl.BlockSpec((1, 2), lambda i: (0, 0)),
      ],
      out_specs=pl.BlockSpec((bnp, 2), lambda i: (i, 0)),
      out_shape=jax.ShapeDtypeStruct((n2, 2), jnp.float32),
  )(h2, msgs2, w2, b2.reshape(1, 128), ws2, bs2.reshape(1, 2))


@functools.cache
def _make_sc_pass(n_src2, n_dst, n_dst_pad, nchunk_tot):
  nchunk_t = nchunk_tot // NS
  rows_per_tile = n_dst_pad // NS
  dr0 = n_dst // NS
  dr_last = n_dst - dr0 * (NS - 1)
  assert nchunk_t % D == 0
  mesh = plsc.VectorSubcoreMesh(core_axis_name="c", subcore_axis_name="s")

  @functools.partial(
      pl.kernel,
      out_type=jax.ShapeDtypeStruct((n_dst, 64), jnp.float32),
      mesh=mesh,
      scratch_types=[
          pltpu.VMEM_SHARED((n_dst_pad, 32), jnp.float32),
          pltpu.VMEM((D, 2, K), jnp.int32),
          pltpu.VMEM((NBUF, K, 32), jnp.float32),
          [pltpu.SemaphoreType.DMA] * D,
          [pltpu.SemaphoreType.DMA] * NBUF,
      ],
      compiler_params=pltpu.CompilerParams(use_tc_tiling_on_sc=False),
  )
  def sc_pass(t_hbm, sidx_hbm, didx_hbm, zeros_hbm, out_hbm, accum, idx_v,
              rows_v, isem, gsem):
    c = lax.axis_index("c")
    s = lax.axis_index("s")
    row0 = s * nchunk_t

    def _ifetch(row, u):
      pltpu.async_copy(sidx_hbm.at[row], idx_v.at[u].at[0], isem[u])
      pltpu.async_copy(didx_hbm.at[row], idx_v.at[u].at[1], isem[u])

    def _iwait(row, u):
      pltpu.make_async_copy(sidx_hbm.at[row], idx_v.at[u].at[0], isem[u]).wait()
      pltpu.make_async_copy(didx_hbm.at[row], idx_v.at[u].at[1], isem[u]).wait()
    t_half = t_hbm.at[pl.ds(c, n_src2 - 1)]

    pltpu.sync_copy(zeros_hbm.at[pl.ds(s * rows_per_tile, rows_per_tile)],
                    accum.at[pl.ds(s * rows_per_tile, rows_per_tile)])

    for u in range(D):
      _ifetch(row0 + u, u)
    for u in range(NBUF):
      _iwait(row0 + u, u)
      pltpu.async_copy(t_half.at[idx_v.at[u].at[0]], rows_v.at[u], gsem[u])

    plsc.subcore_barrier()

    def inner(jj, carry):
      base = jj * D
      for u in range(D):
        j = base + u
        b = u % NBUF
        un = (u + NBUF) % D
        pltpu.make_async_copy(
            t_half.at[idx_v.at[u].at[0]], rows_v.at[b], gsem[b]).wait()
        pltpu.sync_copy(rows_v.at[b], accum.at[idx_v.at[u].at[1]], add=True)
        @pl.when(j + D < nchunk_t)
        def _refill():
          _ifetch(row0 + j + D, u)
        @pl.when(j + NBUF < nchunk_t)
        def _issue():
          _iwait(row0 + j + NBUF, un)
          pltpu.async_copy(
              t_half.at[idx_v.at[un].at[0]], rows_v.at[b], gsem[b])
      return carry
    lax.fori_loop(0, nchunk_t // D, inner, 0)

    plsc.subcore_barrier()

    def _drain(sl):
      @pl.when(c == 0)
      def _d0():
        pltpu.sync_copy(accum.at[sl], out_hbm.at[sl, pl.ds(0, 32)])

      @pl.when(c == 1)
      def _d1():
        pltpu.sync_copy(accum.at[sl], out_hbm.at[sl, pl.ds(32, 32)])

    if dr0 * NS == n_dst:
      _drain(pl.ds(s * dr0, dr0))
    else:
      @pl.when(s < NS - 1)
      def _not_last():
        _drain(pl.ds(s * dr0, dr0))

      @pl.when(s == NS - 1)
      def _last():
        _drain(pl.ds((NS - 1) * dr0, dr_last))

  return sc_pass


def kernel(var_feat, constr_feat, edge_index_var_to_constr,
           W_var, b_var, W_constr, b_constr,
           W_v2c, b_v2c, W_c2v, b_c2v, W_score, b_score):
  v = var_feat.shape[0]
  cn = constr_feat.shape[0]
  e = edge_index_var_to_constr.shape[1]

  v_pad = _ceil_to(v + 1, NS * K)
  c_pad = _ceil_to(cn + 1, NS * K)
  e_pad = _ceil_to(e + 1, NS * K * D)
  nchunk_tot = e_pad // K

  eidx = edge_index_var_to_constr.astype(jnp.int32)
  vidx, cidx = eidx[0], eidx[1]
  npad = e_pad - e
  ar = jnp.arange(npad, dtype=jnp.int32)
  sidx_v2c = (2 * jnp.concatenate([vidx, ar % v])).reshape(nchunk_tot, K)
  didx_v2c = jnp.concatenate([cidx, cn + ar % (c_pad - cn)]).reshape(nchunk_tot, K)
  sidx_c2v = (2 * jnp.concatenate([cidx, ar % cn])).reshape(nchunk_tot, K)
  didx_c2v = jnp.concatenate([vidx, v + ar % (v_pad - v)]).reshape(nchunk_tot, K)

  v2c = _make_sc_pass(2 * v, cn, c_pad, nchunk_tot)
  c2v = _make_sc_pass(2 * cn, v, v_pad, nchunk_tot)

  eye2 = jnp.eye(2, dtype=jnp.float32)
  w_var2 = jnp.kron(eye2, W_var)
  w_constr2 = jnp.kron(eye2, W_constr)
  w_v2c2 = jnp.kron(eye2, W_v2c)
  w_c2v2 = jnp.kron(eye2, W_c2v)
  ws2 = jnp.kron(eye2, W_score)
  b_var2 = jnp.tile(b_var, 2)
  b_constr2 = jnp.tile(b_constr, 2)
  b_v2c2 = jnp.tile(b_v2c, 2)
  b_c2v2 = jnp.tile(b_c2v, 2)
  bs2 = jnp.tile(b_score, 2)

  zeros = jnp.zeros((v_pad, 32), jnp.float32)

  h_var = _embed(var_feat.reshape(v // 2, 256), w_var2, b_var2, 5000)
  h_constr = _embed(constr_feat.reshape(cn // 2, 256), w_constr2, b_constr2,
                    cn // 2)

  rounds = 3
  for r in range(rounds):
    msgs_c = v2c(h_var.reshape(2 * v, 32), sidx_v2c, didx_v2c, zeros)
    h_constr = _update(h_constr, msgs_c.reshape(cn // 2, 128),
                       w_v2c2, b_v2c2, cn // 2)
    msgs_v = c2v(h_constr.reshape(2 * cn, 32), sidx_c2v, didx_c2v, zeros)
    if r < rounds - 1:
      h_var = _update(h_var, msgs_v.reshape(v // 2, 128),
                      w_c2v2, b_c2v2, 5000)
    else:
      scores = _score(h_var, msgs_v.reshape(v // 2, 128),
                      w_c2v2, b_c2v2, ws2, bs2, 5000)

  return scores.reshape(-1)
